# Initial kernel scaffold; baseline (speedup 1.0000x reference)
import jax
import jax.numpy as jnp
from jax.experimental import pallas as pl


def kernel(x, emb_weight):
    raise NotImplementedError("write your pallas kernel here")


# speedup vs baseline: 1.5762x; 1.5762x over previous
import functools

import jax
import jax.numpy as jnp
from jax import lax
from jax.experimental import pallas as pl
from jax.experimental.pallas import tpu as pltpu
from jax.experimental.pallas import tpu_sc as plsc

BATCH = 16384
FIELDS = 26
EMBED = 32
TOTAL = BATCH * FIELDS
NC = 2
NS = 16
NW = NC * NS
B_PER_W = TOTAL // NW
CHUNK = 1024
NCHUNK = B_PER_W // CHUNK
NBUF = 3


def _emb_body(idx_hbm, table_hbm, out_hbm, idx_v, rows_v, *sems):
    gsems = sems[:NBUF]
    osems = sems[NBUF:]
    wid = lax.axis_index("s") * NC + lax.axis_index("c")
    base = wid * B_PER_W

    pltpu.sync_copy(idx_hbm.at[pl.ds(base, B_PER_W)], idx_v)

    def start_gather(g):
        b = g % NBUF
        return pltpu.async_copy(
            table_hbm.at[idx_v.at[pl.ds(g * CHUNK, CHUNK)]],
            rows_v.at[b], gsems[b])

    gds = [None] * NCHUNK
    ods = [None] * NCHUNK
    for g in range(min(NBUF, NCHUNK)):
        gds[g] = start_gather(g)
    for g in range(NCHUNK):
        b = g % NBUF
        gds[g].wait()
        ods[g] = pltpu.async_copy(
            rows_v.at[b],
            out_hbm.at[pl.ds(base + g * CHUNK, CHUNK)],
            osems[b])
        nxt = g + NBUF
        if nxt < NCHUNK:
            ods[g].wait()
            gds[nxt] = start_gather(nxt)
    for g in range(max(NCHUNK - NBUF, 0), NCHUNK):
        ods[g].wait()


@jax.jit
def kernel(x, emb_weight):
    idx = x.astype(jnp.int32).reshape(TOTAL)
    mesh = plsc.VectorSubcoreMesh(core_axis_name="c", subcore_axis_name="s")
    run = functools.partial(
        pl.kernel,
        out_type=jax.ShapeDtypeStruct((TOTAL, EMBED), jnp.float32),
        mesh=mesh,
        scratch_types=[
            pltpu.VMEM((B_PER_W,), jnp.int32),
            pltpu.VMEM((NBUF, CHUNK, EMBED), jnp.float32),
        ] + [pltpu.SemaphoreType.DMA] * (2 * NBUF),
        compiler_params=pltpu.CompilerParams(use_tc_tiling_on_sc=False),
    )(_emb_body)
    out = run(idx, emb_weight)
    return out.reshape(BATCH, FIELDS, EMBED)
